# baseline (speedup 1.0000x reference)
import functools

import jax
import jax.numpy as jnp
from jax import lax
from jax.experimental import pallas as pl
from jax.experimental.pallas import tpu as pltpu
from jax.experimental.pallas import tpu_sc as plsc

N = 10000
E = 320000
D = 128
NC = 2
NS = 16
NW = NC * NS
RPW = 80
EROWS = NW * RPW
ACC_N = 10240
NPT = 624
NPT1 = ACC_N // NS
CH = 8
NCHUNK = RPW // CH

def _sc_segsum_body(with_counts, *refs):
    if with_counts:
        (u_hbm, srcr, dstr, z128, z1,
         s_out, c_out,
         s_sh, c_sh, src_v, dst_v, rows_v, ones_v, sem) = refs
    else:
        (u_hbm, srcr, dstr, z128,
         s_out,
         s_sh, src_v, dst_v, rows_v, sem) = refs

    cid = lax.axis_index("c")
    sid = lax.axis_index("s")
    wid = cid * NS + sid

    pltpu.sync_copy(z128.at[pl.ds(sid * NPT, NPT)],
                    s_sh.at[pl.ds(sid * NPT, NPT)])
    if with_counts:
        pltpu.sync_copy(z1.at[pl.ds(sid * NPT1, NPT1)],
                        c_sh.at[pl.ds(sid * NPT1, NPT1)])
        for k in range(D // 16):
            ones_v[pl.ds(k * 16, 16)] = jnp.ones((16,), jnp.float32)

    @pl.when(sid == NS - 1)
    def _():
        tail = ACC_N - NS * NPT
        pltpu.sync_copy(z128.at[pl.ds(0, tail)],
                        s_sh.at[pl.ds(NS * NPT, tail)])

    base = wid * RPW
    pltpu.sync_copy(srcr.at[pl.ds(base, CH)], src_v.at[0])
    pltpu.sync_copy(dstr.at[pl.ds(base, CH)], dst_v.at[0])

    plsc.subcore_barrier()

    pltpu.async_copy(u_hbm.at[src_v.at[0].at[0]], rows_v.at[0], sem)

    def step(j, carry):
        c = j // CH
        r = j - c * CH

        @pl.when((r == 0) & (c + 1 < NCHUNK))
        def _():
            pltpu.sync_copy(srcr.at[pl.ds(base + (c + 1) * CH, CH)],
                            src_v.at[(c + 1) % 2])
            pltpu.sync_copy(dstr.at[pl.ds(base + (c + 1) * CH, CH)],
                            dst_v.at[(c + 1) % 2])

        nxt = j + 1

        @pl.when(nxt < RPW)
        def _():
            c2 = nxt // CH
            r2 = nxt - c2 * CH
            pltpu.async_copy(u_hbm.at[src_v.at[c2 % 2].at[r2]],
                             rows_v.at[nxt % 2], sem)

        pltpu.make_async_copy(u_hbm.at[src_v.at[c % 2].at[r]],
                              rows_v.at[j % 2], sem).wait()
        pltpu.sync_copy(rows_v.at[j % 2], s_sh.at[dst_v.at[c % 2].at[r]],
                        add=True)
        if with_counts:
            pltpu.sync_copy(ones_v, c_sh.at[dst_v.at[c % 2].at[r]], add=True)
        return carry

    lax.fori_loop(0, RPW, step, 0)

    plsc.subcore_barrier()

    pltpu.sync_copy(s_sh.at[pl.ds(sid * NPT, NPT)],
                    s_out.at[cid].at[pl.ds(sid * NPT, NPT)])
    if with_counts:
        pltpu.sync_copy(c_sh.at[pl.ds(sid * NPT1, NPT1)],
                        c_out.at[cid].at[pl.ds(sid * NPT1, NPT1)])

    @pl.when(sid == NS - 1)
    def _():
        tail = N - NS * NPT
        pltpu.sync_copy(s_sh.at[pl.ds(NS * NPT, tail)],
                        s_out.at[cid].at[pl.ds(NS * NPT, tail)])


@functools.cache
def _sc_kernels():
    mesh = plsc.VectorSubcoreMesh(
        core_axis_name="c", subcore_axis_name="s",
        num_cores=NC, num_subcores=NS,
    )
    layer1 = functools.partial(
        pl.kernel,
        functools.partial(_sc_segsum_body, True),
        out_type=(
            jax.ShapeDtypeStruct((NC, N, D), jnp.float32),
            jax.ShapeDtypeStruct((NC, ACC_N), jnp.float32),
        ),
        mesh=mesh,
        scratch_types=[
            pltpu.VMEM_SHARED((ACC_N, D), jnp.float32),
            pltpu.VMEM_SHARED((ACC_N,), jnp.float32),
            pltpu.VMEM((2, CH, D), jnp.int32),
            pltpu.VMEM((2, CH, D), jnp.int32),
            pltpu.VMEM((2, D, D), jnp.float32),
            pltpu.VMEM((D,), jnp.float32),
            pltpu.SemaphoreType.DMA,
        ],
    )()
    layer2 = functools.partial(
        pl.kernel,
        functools.partial(_sc_segsum_body, False),
        out_type=jax.ShapeDtypeStruct((NC, N, D), jnp.float32),
        mesh=mesh,
        scratch_types=[
            pltpu.VMEM_SHARED((ACC_N, D), jnp.float32),
            pltpu.VMEM((2, CH, D), jnp.int32),
            pltpu.VMEM((2, CH, D), jnp.int32),
            pltpu.VMEM((2, D, D), jnp.float32),
            pltpu.SemaphoreType.DMA,
        ],
    )()
    return layer1, layer2


_RB = 1000


def _tc_pre_body(x_ref, wl_ref, b_ref, wr_ref, u_ref, v_ref):
    x = x_ref[...]
    u_ref[...] = jnp.dot(x, wl_ref[...].T, preferred_element_type=jnp.float32)
    v_ref[...] = (jnp.dot(x, wr_ref[...].T, preferred_element_type=jnp.float32)
                  + b_ref[...])


def _tc_mid_body(s_ref, c_ref, v1_ref, wl_ref, b_ref, wr_ref, u2_ref, v2_ref):
    cnt = c_ref[...]
    inv = 1.0 / jnp.maximum(cnt, 1.0)
    h = jnp.maximum((s_ref[0] + s_ref[1]) * inv + v1_ref[...], 0.0)
    u2_ref[...] = jnp.dot(h, wl_ref[...].T, preferred_element_type=jnp.float32)
    v2_ref[...] = (jnp.dot(h, wr_ref[...].T, preferred_element_type=jnp.float32)
                   + b_ref[...])


def _tc_post_body(s_ref, c_ref, v2_ref, out_ref):
    cnt = c_ref[...]
    inv = 1.0 / jnp.maximum(cnt, 1.0)
    out_ref[...] = (s_ref[0] + s_ref[1]) * inv + v2_ref[...]


def _full(shape):
    return pl.BlockSpec(shape, lambda i: (0,) * len(shape))


def _rows(shape):
    if len(shape) == 3:
        return pl.BlockSpec(shape, lambda i: (0, i, 0))
    return pl.BlockSpec(shape, lambda i: (i, 0))


_tc_pre = pl.pallas_call(
    _tc_pre_body,
    grid=(N // _RB,),
    in_specs=[_rows((_RB, D)), _full((D, D)), _full((1, D)), _full((D, D))],
    out_specs=[_rows((_RB, D)), _rows((_RB, D))],
    out_shape=(jax.ShapeDtypeStruct((N, D), jnp.float32),
               jax.ShapeDtypeStruct((N, D), jnp.float32)),
)

_tc_mid = pl.pallas_call(
    _tc_mid_body,
    grid=(N // _RB,),
    in_specs=[_rows((NC, _RB, D)), _rows((_RB, 1)), _rows((_RB, D)),
              _full((D, D)), _full((1, D)), _full((D, D))],
    out_specs=[_rows((_RB, D)), _rows((_RB, D))],
    out_shape=(jax.ShapeDtypeStruct((N, D), jnp.float32),
               jax.ShapeDtypeStruct((N, D), jnp.float32)),
)

_tc_post = pl.pallas_call(
    _tc_post_body,
    grid=(N // _RB,),
    in_specs=[_rows((NC, _RB, D)), _rows((_RB, 1)), _rows((_RB, D))],
    out_specs=_rows((_RB, D)),
    out_shape=jax.ShapeDtypeStruct((N, D), jnp.float32),
)


@jax.jit
def kernel(x, edge_index, W_l1, b_l1, W_r1, W_l2, b_l2, W_r2):
    pad = EROWS * D - E
    srcr = jnp.concatenate(
        [edge_index[0].astype(jnp.int32), jnp.zeros((pad,), jnp.int32)]
    ).reshape(EROWS, D)
    dstr = jnp.concatenate(
        [edge_index[1].astype(jnp.int32), N + jnp.arange(pad, dtype=jnp.int32) % (ACC_N - N)]
    ).reshape(EROWS, D)
    z128 = jnp.zeros((N, D), jnp.float32)
    z1 = jnp.zeros((ACC_N,), jnp.float32)

    sc_layer1, sc_layer2 = _sc_kernels()
    u1, v1 = _tc_pre(x, W_l1, b_l1.reshape(1, D), W_r1)
    s1, cpart = sc_layer1(u1, srcr, dstr, z128, z1)
    cnt_t = (cpart[0, :N] + cpart[1, :N]).reshape(N, 1)
    u2, v2 = _tc_mid(s1, cnt_t, v1, W_l2, b_l2.reshape(1, D), W_r2)
    s2 = sc_layer2(u2, srcr, dstr, z128)
    return _tc_post(s2, cnt_t, v2)


# speedup vs baseline: 1.0362x; 1.0362x over previous
import functools

import jax
import jax.numpy as jnp
from jax import lax
from jax.experimental import pallas as pl
from jax.experimental.pallas import tpu as pltpu
from jax.experimental.pallas import tpu_sc as plsc

N = 10000
E = 320000
D = 128
NC = 2
NS = 16
NW = NC * NS
RPW0 = 128
RPW1 = 32
EROWS = NS * (RPW0 + RPW1)
ACC_N = 10240
NPT = 624
NPT1 = ACC_N // NS
CH = 8

def _sc_segsum_body(with_counts, *refs):
    if with_counts:
        (u_hbm, srcr, dstr, z128, z1,
         s_out, c_out,
         s_sh, c_sh, src_v, dst_v, rows_v, ones_v, sem) = refs
    else:
        (u_hbm, srcr, dstr, z128,
         s_out,
         s_sh, src_v, dst_v, rows_v, sem) = refs

    cid = lax.axis_index("c")
    sid = lax.axis_index("s")
    wid = cid * NS + sid

    pltpu.sync_copy(z128.at[pl.ds(sid * NPT, NPT)],
                    s_sh.at[pl.ds(sid * NPT, NPT)])
    if with_counts:
        pltpu.sync_copy(z1.at[pl.ds(sid * NPT1, NPT1)],
                        c_sh.at[pl.ds(sid * NPT1, NPT1)])
        for k in range(D // 16):
            ones_v[pl.ds(k * 16, 16)] = jnp.ones((16,), jnp.float32)

    @pl.when(sid == NS - 1)
    def _():
        tail = ACC_N - NS * NPT
        pltpu.sync_copy(z128.at[pl.ds(0, tail)],
                        s_sh.at[pl.ds(NS * NPT, tail)])

    rpw = jnp.where(cid == 0, RPW0, RPW1)
    base = jnp.where(cid == 0, sid * RPW0, NS * RPW0 + sid * RPW1)
    pltpu.sync_copy(srcr.at[pl.ds(base, CH)], src_v.at[0])
    pltpu.sync_copy(dstr.at[pl.ds(base, CH)], dst_v.at[0])

    plsc.subcore_barrier()

    pltpu.async_copy(u_hbm.at[src_v.at[0].at[0]], rows_v.at[0], sem)

    def step(j, carry):
        c = j // CH
        r = j - c * CH

        @pl.when((r == 0) & ((c + 1) * CH < rpw))
        def _():
            pltpu.sync_copy(srcr.at[pl.ds(base + (c + 1) * CH, CH)],
                            src_v.at[(c + 1) % 2])
            pltpu.sync_copy(dstr.at[pl.ds(base + (c + 1) * CH, CH)],
                            dst_v.at[(c + 1) % 2])

        nxt = j + 1

        @pl.when(nxt < rpw)
        def _():
            c2 = nxt // CH
            r2 = nxt - c2 * CH
            pltpu.async_copy(u_hbm.at[src_v.at[c2 % 2].at[r2]],
                             rows_v.at[nxt % 2], sem)

        pltpu.make_async_copy(u_hbm.at[src_v.at[c % 2].at[r]],
                              rows_v.at[j % 2], sem).wait()
        pltpu.sync_copy(rows_v.at[j % 2], s_sh.at[dst_v.at[c % 2].at[r]],
                        add=True)
        if with_counts:
            pltpu.sync_copy(ones_v, c_sh.at[dst_v.at[c % 2].at[r]], add=True)
        return carry

    lax.fori_loop(0, rpw, step, 0)

    plsc.subcore_barrier()

    pltpu.sync_copy(s_sh.at[pl.ds(sid * NPT, NPT)],
                    s_out.at[cid].at[pl.ds(sid * NPT, NPT)])
    if with_counts:
        pltpu.sync_copy(c_sh.at[pl.ds(sid * NPT1, NPT1)],
                        c_out.at[cid].at[pl.ds(sid * NPT1, NPT1)])

    @pl.when(sid == NS - 1)
    def _():
        tail = N - NS * NPT
        pltpu.sync_copy(s_sh.at[pl.ds(NS * NPT, tail)],
                        s_out.at[cid].at[pl.ds(NS * NPT, tail)])


@functools.cache
def _sc_kernels():
    mesh = plsc.VectorSubcoreMesh(
        core_axis_name="c", subcore_axis_name="s",
        num_cores=NC, num_subcores=NS,
    )
    layer1 = functools.partial(
        pl.kernel,
        functools.partial(_sc_segsum_body, True),
        out_type=(
            jax.ShapeDtypeStruct((NC, N, D), jnp.float32),
            jax.ShapeDtypeStruct((NC, ACC_N), jnp.float32),
        ),
        mesh=mesh,
        scratch_types=[
            pltpu.VMEM_SHARED((ACC_N, D), jnp.float32),
            pltpu.VMEM_SHARED((ACC_N,), jnp.float32),
            pltpu.VMEM((2, CH, D), jnp.int32),
            pltpu.VMEM((2, CH, D), jnp.int32),
            pltpu.VMEM((2, D, D), jnp.float32),
            pltpu.VMEM((D,), jnp.float32),
            pltpu.SemaphoreType.DMA,
        ],
    )()
    layer2 = functools.partial(
        pl.kernel,
        functools.partial(_sc_segsum_body, False),
        out_type=jax.ShapeDtypeStruct((NC, N, D), jnp.float32),
        mesh=mesh,
        scratch_types=[
            pltpu.VMEM_SHARED((ACC_N, D), jnp.float32),
            pltpu.VMEM((2, CH, D), jnp.int32),
            pltpu.VMEM((2, CH, D), jnp.int32),
            pltpu.VMEM((2, D, D), jnp.float32),
            pltpu.SemaphoreType.DMA,
        ],
    )()
    return layer1, layer2


_RB = 1000


def _tc_pre_body(x_ref, wl_ref, b_ref, wr_ref, u_ref, v_ref):
    x = x_ref[...]
    u_ref[...] = jnp.dot(x, wl_ref[...].T, preferred_element_type=jnp.float32)
    v_ref[...] = (jnp.dot(x, wr_ref[...].T, preferred_element_type=jnp.float32)
                  + b_ref[...])


def _tc_mid_body(s_ref, c_ref, v1_ref, wl_ref, b_ref, wr_ref, u2_ref, v2_ref):
    cnt = c_ref[...]
    inv = 1.0 / jnp.maximum(cnt, 1.0)
    h = jnp.maximum((s_ref[0] + s_ref[1]) * inv + v1_ref[...], 0.0)
    u2_ref[...] = jnp.dot(h, wl_ref[...].T, preferred_element_type=jnp.float32)
    v2_ref[...] = (jnp.dot(h, wr_ref[...].T, preferred_element_type=jnp.float32)
                   + b_ref[...])


def _tc_post_body(s_ref, c_ref, v2_ref, out_ref):
    cnt = c_ref[...]
    inv = 1.0 / jnp.maximum(cnt, 1.0)
    out_ref[...] = (s_ref[0] + s_ref[1]) * inv + v2_ref[...]


def _full(shape):
    return pl.BlockSpec(shape, lambda i: (0,) * len(shape))


def _rows(shape):
    if len(shape) == 3:
        return pl.BlockSpec(shape, lambda i: (0, i, 0))
    return pl.BlockSpec(shape, lambda i: (i, 0))


_tc_pre = pl.pallas_call(
    _tc_pre_body,
    grid=(N // _RB,),
    in_specs=[_rows((_RB, D)), _full((D, D)), _full((1, D)), _full((D, D))],
    out_specs=[_rows((_RB, D)), _rows((_RB, D))],
    out_shape=(jax.ShapeDtypeStruct((N, D), jnp.float32),
               jax.ShapeDtypeStruct((N, D), jnp.float32)),
)

_tc_mid = pl.pallas_call(
    _tc_mid_body,
    grid=(N // _RB,),
    in_specs=[_rows((NC, _RB, D)), _rows((_RB, 1)), _rows((_RB, D)),
              _full((D, D)), _full((1, D)), _full((D, D))],
    out_specs=[_rows((_RB, D)), _rows((_RB, D))],
    out_shape=(jax.ShapeDtypeStruct((N, D), jnp.float32),
               jax.ShapeDtypeStruct((N, D), jnp.float32)),
)

_tc_post = pl.pallas_call(
    _tc_post_body,
    grid=(N // _RB,),
    in_specs=[_rows((NC, _RB, D)), _rows((_RB, 1)), _rows((_RB, D))],
    out_specs=_rows((_RB, D)),
    out_shape=jax.ShapeDtypeStruct((N, D), jnp.float32),
)


@jax.jit
def kernel(x, edge_index, W_l1, b_l1, W_r1, W_l2, b_l2, W_r2):
    pad = EROWS * D - E
    srcr = jnp.concatenate(
        [edge_index[0].astype(jnp.int32), jnp.zeros((pad,), jnp.int32)]
    ).reshape(EROWS, D)
    dstr = jnp.concatenate(
        [edge_index[1].astype(jnp.int32), N + jnp.arange(pad, dtype=jnp.int32) % (ACC_N - N)]
    ).reshape(EROWS, D)
    z128 = jnp.zeros((N, D), jnp.float32)
    z1 = jnp.zeros((ACC_N,), jnp.float32)

    sc_layer1, sc_layer2 = _sc_kernels()
    u1, v1 = _tc_pre(x, W_l1, b_l1.reshape(1, D), W_r1)
    s1, cpart = sc_layer1(u1, srcr, dstr, z128, z1)
    cnt_t = (cpart[0, :N] + cpart[1, :N]).reshape(N, 1)
    u2, v2 = _tc_mid(s1, cnt_t, v1, W_l2, b_l2.reshape(1, D), W_r2)
    s2 = sc_layer2(u2, srcr, dstr, z128)
    return _tc_post(s2, cnt_t, v2)


# speedup vs baseline: 3.3421x; 3.2255x over previous
import functools

import jax
import jax.numpy as jnp
from jax import lax
from jax.experimental import pallas as pl
from jax.experimental.pallas import tpu as pltpu
from jax.experimental.pallas import tpu_sc as plsc

N = 10000
E = 320000
D = 128
NC = 2
NS = 16
NW = NC * NS
RPW = 80
EROWS = NW * RPW
ACC_N = 10240
NPT = 624
NPT1 = ACC_N // NS
CH = 8


def _sc_segsum_body(with_counts, *refs):
    if with_counts:
        (u_hbm, srcr, dstr, z128, z1,
         s_out, c_out,
         s_sh, c_sh, src_v, dst_v, rows_v, ones_v, sem) = refs
    else:
        (u_hbm, srcr, dstr, z128,
         s_out,
         s_sh, src_v, dst_v, rows_v, sem) = refs

    cid = lax.axis_index("c")
    sid = lax.axis_index("s")
    wid = cid * NS + sid

    pltpu.sync_copy(z128.at[pl.ds(sid * NPT, NPT)],
                    s_sh.at[pl.ds(sid * NPT, NPT)])
    if with_counts:
        pltpu.sync_copy(z1.at[pl.ds(sid * NPT1, NPT1)],
                        c_sh.at[pl.ds(sid * NPT1, NPT1)])
        for k in range(D // 16):
            ones_v[pl.ds(k * 16, 16)] = jnp.ones((16,), jnp.float32)

    @pl.when(sid == NS - 1)
    def _():
        tail = ACC_N - NS * NPT
        pltpu.sync_copy(z128.at[pl.ds(0, tail)],
                        s_sh.at[pl.ds(NS * NPT, tail)])

    base = wid * RPW
    pltpu.sync_copy(srcr.at[pl.ds(base, CH)], src_v.at[0])
    pltpu.sync_copy(dstr.at[pl.ds(base, CH)], dst_v.at[0])

    plsc.subcore_barrier()

    pltpu.async_copy(u_hbm.at[src_v.at[0].at[0]], rows_v.at[0], sem)

    def step(j, carry):
        c = j // CH
        r = j - c * CH

        @pl.when((r == 0) & ((c + 1) * CH < RPW))
        def _():
            pltpu.sync_copy(srcr.at[pl.ds(base + (c + 1) * CH, CH)],
                            src_v.at[(c + 1) % 2])
            pltpu.sync_copy(dstr.at[pl.ds(base + (c + 1) * CH, CH)],
                            dst_v.at[(c + 1) % 2])

        nxt = j + 1

        @pl.when(nxt < RPW)
        def _():
            c2 = nxt // CH
            r2 = nxt - c2 * CH
            pltpu.async_copy(u_hbm.at[src_v.at[c2 % 2].at[r2]],
                             rows_v.at[nxt % 2], sem)

        pltpu.make_async_copy(u_hbm.at[src_v.at[c % 2].at[r]],
                              rows_v.at[j % 2], sem).wait()
        pltpu.sync_copy(rows_v.at[j % 2], s_sh.at[dst_v.at[c % 2].at[r]],
                        add=True)
        if with_counts:
            pltpu.sync_copy(ones_v, c_sh.at[dst_v.at[c % 2].at[r]], add=True)
        return carry

    lax.fori_loop(0, RPW, step, 0)

    plsc.subcore_barrier()

    pltpu.sync_copy(s_sh.at[pl.ds(sid * NPT, NPT)],
                    s_out.at[cid].at[pl.ds(sid * NPT, NPT)])
    if with_counts:
        pltpu.sync_copy(c_sh.at[pl.ds(sid * NPT1, NPT1)],
                        c_out.at[cid].at[pl.ds(sid * NPT1, NPT1)])

    @pl.when(sid == NS - 1)
    def _():
        tail = N - NS * NPT
        pltpu.sync_copy(s_sh.at[pl.ds(NS * NPT, tail)],
                        s_out.at[cid].at[pl.ds(NS * NPT, tail)])


@functools.cache
def _sc_kernels():
    mesh = plsc.VectorSubcoreMesh(
        core_axis_name="c", subcore_axis_name="s",
        num_cores=NC, num_subcores=NS,
    )
    layer1 = functools.partial(
        pl.kernel,
        functools.partial(_sc_segsum_body, True),
        out_type=(
            jax.ShapeDtypeStruct((NC, N, D), jnp.float32),
            jax.ShapeDtypeStruct((NC, ACC_N), jnp.float32),
        ),
        mesh=mesh,
        scratch_types=[
            pltpu.VMEM_SHARED((ACC_N, D), jnp.float32),
            pltpu.VMEM_SHARED((ACC_N,), jnp.float32),
            pltpu.VMEM((2, CH, D), jnp.int32),
            pltpu.VMEM((2, CH, D), jnp.int32),
            pltpu.VMEM((2, D, D), jnp.float32),
            pltpu.VMEM((D,), jnp.float32),
            pltpu.SemaphoreType.DMA,
        ],
    )()
    layer2 = functools.partial(
        pl.kernel,
        functools.partial(_sc_segsum_body, False),
        out_type=jax.ShapeDtypeStruct((NC, N, D), jnp.float32),
        mesh=mesh,
        scratch_types=[
            pltpu.VMEM_SHARED((ACC_N, D), jnp.float32),
            pltpu.VMEM((2, CH, D), jnp.int32),
            pltpu.VMEM((2, CH, D), jnp.int32),
            pltpu.VMEM((2, D, D), jnp.float32),
            pltpu.SemaphoreType.DMA,
        ],
    )()
    return layer1, layer2


_RB = 1000


def _tc_pre_body(x_ref, wl_ref, b_ref, wr_ref, u_ref, v_ref):
    x = x_ref[...]
    u_ref[...] = jnp.dot(x, wl_ref[...].T, preferred_element_type=jnp.float32)
    v_ref[...] = (jnp.dot(x, wr_ref[...].T, preferred_element_type=jnp.float32)
                  + b_ref[...])


def _tc_mid_body(s_ref, c_ref, v1_ref, wl_ref, b_ref, wr_ref, u2_ref, v2_ref):
    inv = 1.0 / jnp.maximum(c_ref[...], 1.0)
    h = jnp.maximum((s_ref[0] + s_ref[1]) * inv + v1_ref[...], 0.0)
    u2_ref[...] = jnp.dot(h, wl_ref[...].T, preferred_element_type=jnp.float32)
    v2_ref[...] = (jnp.dot(h, wr_ref[...].T, preferred_element_type=jnp.float32)
                   + b_ref[...])


def _tc_post_body(s_ref, c_ref, v2_ref, out_ref):
    inv = 1.0 / jnp.maximum(c_ref[...], 1.0)
    out_ref[...] = (s_ref[0] + s_ref[1]) * inv + v2_ref[...]


def _full(shape):
    return pl.BlockSpec(shape, lambda i: (0,) * len(shape))


def _rows(shape):
    if len(shape) == 3:
        return pl.BlockSpec(shape, lambda i: (0, i, 0))
    return pl.BlockSpec(shape, lambda i: (i, 0))


_tc_pre = pl.pallas_call(
    _tc_pre_body,
    grid=(N // _RB,),
    in_specs=[_rows((_RB, D)), _full((D, D)), _full((1, D)), _full((D, D))],
    out_specs=[_rows((_RB, D)), _rows((_RB, D))],
    out_shape=(jax.ShapeDtypeStruct((N, D), jnp.float32),
               jax.ShapeDtypeStruct((N, D), jnp.float32)),
)

_tc_mid = pl.pallas_call(
    _tc_mid_body,
    grid=(N // _RB,),
    in_specs=[_rows((NC, _RB, D)), _rows((_RB, 1)), _rows((_RB, D)),
              _full((D, D)), _full((1, D)), _full((D, D))],
    out_specs=[_rows((_RB, D)), _rows((_RB, D))],
    out_shape=(jax.ShapeDtypeStruct((N, D), jnp.float32),
               jax.ShapeDtypeStruct((N, D), jnp.float32)),
)

_tc_post = pl.pallas_call(
    _tc_post_body,
    grid=(N // _RB,),
    in_specs=[_rows((NC, _RB, D)), _rows((_RB, 1)), _rows((_RB, D))],
    out_specs=_rows((_RB, D)),
    out_shape=jax.ShapeDtypeStruct((N, D), jnp.float32),
)


@jax.jit
def kernel(x, edge_index, W_l1, b_l1, W_r1, W_l2, b_l2, W_r2):
    pad = EROWS * D - E
    pidx = jnp.arange(pad, dtype=jnp.int32)
    srcr = jnp.concatenate(
        [edge_index[0].astype(jnp.int32), pidx % N]
    ).reshape(EROWS, D)
    dstr = jnp.concatenate(
        [edge_index[1].astype(jnp.int32), N + pidx % (ACC_N - N)]
    ).reshape(EROWS, D)
    z128 = jnp.zeros((N, D), jnp.float32)
    z1 = jnp.zeros((ACC_N,), jnp.float32)

    sc_layer1, sc_layer2 = _sc_kernels()
    u1, v1 = _tc_pre(x, W_l1, b_l1.reshape(1, D), W_r1)
    s1, cpart = sc_layer1(u1, srcr, dstr, z128, z1)
    cnt_t = (cpart[0, :N] + cpart[1, :N]).reshape(N, 1)
    u2, v2 = _tc_mid(s1, cnt_t, v1, W_l2, b_l2.reshape(1, D), W_r2)
    s2 = sc_layer2(u2, srcr, dstr, z128)
    return _tc_post(s2, cnt_t, v2)


# speedup vs baseline: 3.6401x; 1.0892x over previous
import functools

import jax
import jax.numpy as jnp
from jax import lax
from jax.experimental import pallas as pl
from jax.experimental.pallas import tpu as pltpu
from jax.experimental.pallas import tpu_sc as plsc

N = 10000
E = 320000
D = 128
NC = 2
NS = 16
NW = NC * NS
RPW = 80
EROWS = NW * RPW
ACC_N = 10240
NPT = 624
NPT1 = ACC_N // NS
CH = 8


def _sc_segsum_body(with_counts, *refs):
    if with_counts:
        (u_hbm, srcr, dstr,
         s_out, c_out,
         s_sh, c_sh, src_v, dst_v, rows_v, ones_v, zc_v, sem, isem) = refs
    else:
        (u_hbm, srcr, dstr,
         s_out,
         s_sh, src_v, dst_v, rows_v, sem, isem) = refs

    cid = lax.axis_index("c")
    sid = lax.axis_index("s")
    wid = cid * NS + sid

    zero16 = jnp.zeros((16,), jnp.float32)

    def zrow(i, carry):
        for k in range(D // 16):
            rows_v[0, i, pl.ds(k * 16, 16)] = zero16
        return carry

    lax.fori_loop(0, D, zrow, 0)
    for b in range(NPT // D):
        pltpu.sync_copy(rows_v.at[0],
                        s_sh.at[pl.ds(sid * NPT + b * D, D)])
    rem = NPT - (NPT // D) * D
    pltpu.sync_copy(rows_v.at[0].at[pl.ds(0, rem)],
                    s_sh.at[pl.ds(sid * NPT + (NPT // D) * D, rem)])
    if with_counts:
        for k in range(NPT1 // 16):
            zc_v[pl.ds(k * 16, 16)] = zero16
        pltpu.sync_copy(zc_v, c_sh.at[pl.ds(sid * NPT1, NPT1)])
        for k in range(D // 16):
            ones_v[pl.ds(k * 16, 16)] = jnp.ones((16,), jnp.float32)

    @pl.when(sid == NS - 1)
    def _():
        tail = ACC_N - NS * NPT
        for b in range(tail // D):
            pltpu.sync_copy(rows_v.at[0],
                            s_sh.at[pl.ds(NS * NPT + b * D, D)])

    base = wid * RPW
    pltpu.sync_copy(srcr.at[pl.ds(base, CH)], src_v.at[0])
    pltpu.sync_copy(dstr.at[pl.ds(base, CH)], dst_v.at[0])

    plsc.subcore_barrier()

    pltpu.async_copy(u_hbm.at[src_v.at[0].at[0]], rows_v.at[0], sem)

    def step(j, carry):
        c = j // CH
        r = j - c * CH

        @pl.when((r == 0) & ((c + 1) * CH < RPW))
        def _():
            pltpu.async_copy(srcr.at[pl.ds(base + (c + 1) * CH, CH)],
                             src_v.at[(c + 1) % 2], isem)
            pltpu.async_copy(dstr.at[pl.ds(base + (c + 1) * CH, CH)],
                             dst_v.at[(c + 1) % 2], isem)

        @pl.when((r == CH - 1) & ((c + 1) * CH < RPW))
        def _():
            pltpu.make_async_copy(srcr.at[pl.ds(base, CH)],
                                  src_v.at[(c + 1) % 2], isem).wait()
            pltpu.make_async_copy(dstr.at[pl.ds(base, CH)],
                                  dst_v.at[(c + 1) % 2], isem).wait()

        nxt = j + 1

        @pl.when(nxt < RPW)
        def _():
            c2 = nxt // CH
            r2 = nxt - c2 * CH
            pltpu.async_copy(u_hbm.at[src_v.at[c2 % 2].at[r2]],
                             rows_v.at[nxt % 2], sem)

        pltpu.make_async_copy(u_hbm.at[src_v.at[c % 2].at[r]],
                              rows_v.at[j % 2], sem).wait()
        pltpu.sync_copy(rows_v.at[j % 2], s_sh.at[dst_v.at[c % 2].at[r]],
                        add=True)
        if with_counts:
            pltpu.sync_copy(ones_v, c_sh.at[dst_v.at[c % 2].at[r]], add=True)
        return carry

    lax.fori_loop(0, RPW, step, 0)

    plsc.subcore_barrier()

    pltpu.sync_copy(s_sh.at[pl.ds(sid * NPT, NPT)],
                    s_out.at[cid].at[pl.ds(sid * NPT, NPT)])
    if with_counts:
        pltpu.sync_copy(c_sh.at[pl.ds(sid * NPT1, NPT1)],
                        c_out.at[cid].at[pl.ds(sid * NPT1, NPT1)])

    @pl.when(sid == NS - 1)
    def _():
        tail = N - NS * NPT
        pltpu.sync_copy(s_sh.at[pl.ds(NS * NPT, tail)],
                        s_out.at[cid].at[pl.ds(NS * NPT, tail)])


@functools.cache
def _sc_kernels():
    mesh = plsc.VectorSubcoreMesh(
        core_axis_name="c", subcore_axis_name="s",
        num_cores=NC, num_subcores=NS,
    )
    layer1 = functools.partial(
        pl.kernel,
        functools.partial(_sc_segsum_body, True),
        out_type=(
            jax.ShapeDtypeStruct((NC, N, D), jnp.float32),
            jax.ShapeDtypeStruct((NC, ACC_N), jnp.float32),
        ),
        mesh=mesh,
        scratch_types=[
            pltpu.VMEM_SHARED((ACC_N, D), jnp.float32),
            pltpu.VMEM_SHARED((ACC_N,), jnp.float32),
            pltpu.VMEM((2, CH, D), jnp.int32),
            pltpu.VMEM((2, CH, D), jnp.int32),
            pltpu.VMEM((2, D, D), jnp.float32),
            pltpu.VMEM((D,), jnp.float32),
            pltpu.VMEM((NPT1,), jnp.float32),
            pltpu.SemaphoreType.DMA,
            pltpu.SemaphoreType.DMA,
        ],
    )()
    layer2 = functools.partial(
        pl.kernel,
        functools.partial(_sc_segsum_body, False),
        out_type=jax.ShapeDtypeStruct((NC, N, D), jnp.float32),
        mesh=mesh,
        scratch_types=[
            pltpu.VMEM_SHARED((ACC_N, D), jnp.float32),
            pltpu.VMEM((2, CH, D), jnp.int32),
            pltpu.VMEM((2, CH, D), jnp.int32),
            pltpu.VMEM((2, D, D), jnp.float32),
            pltpu.SemaphoreType.DMA,
            pltpu.SemaphoreType.DMA,
        ],
    )()
    return layer1, layer2


_RB = 1000


def _tc_pre_body(x_ref, wl_ref, b_ref, wr_ref, u_ref, v_ref):
    x = x_ref[...]
    u_ref[...] = jnp.dot(x, wl_ref[...].T, preferred_element_type=jnp.float32)
    v_ref[...] = (jnp.dot(x, wr_ref[...].T, preferred_element_type=jnp.float32)
                  + b_ref[...])


def _tc_mid_body(s_ref, c_ref, v1_ref, wl_ref, b_ref, wr_ref, u2_ref, v2_ref):
    inv = 1.0 / jnp.maximum(c_ref[...], 1.0)
    h = jnp.maximum((s_ref[0] + s_ref[1]) * inv + v1_ref[...], 0.0)
    u2_ref[...] = jnp.dot(h, wl_ref[...].T, preferred_element_type=jnp.float32)
    v2_ref[...] = (jnp.dot(h, wr_ref[...].T, preferred_element_type=jnp.float32)
                   + b_ref[...])


def _tc_post_body(s_ref, c_ref, v2_ref, out_ref):
    inv = 1.0 / jnp.maximum(c_ref[...], 1.0)
    out_ref[...] = (s_ref[0] + s_ref[1]) * inv + v2_ref[...]


def _full(shape):
    return pl.BlockSpec(shape, lambda i: (0,) * len(shape))


def _rows(shape):
    if len(shape) == 3:
        return pl.BlockSpec(shape, lambda i: (0, i, 0))
    return pl.BlockSpec(shape, lambda i: (i, 0))


_tc_pre = pl.pallas_call(
    _tc_pre_body,
    grid=(N // _RB,),
    in_specs=[_rows((_RB, D)), _full((D, D)), _full((1, D)), _full((D, D))],
    out_specs=[_rows((_RB, D)), _rows((_RB, D))],
    out_shape=(jax.ShapeDtypeStruct((N, D), jnp.float32),
               jax.ShapeDtypeStruct((N, D), jnp.float32)),
)

_tc_mid = pl.pallas_call(
    _tc_mid_body,
    grid=(N // _RB,),
    in_specs=[_rows((NC, _RB, D)), _rows((_RB, 1)), _rows((_RB, D)),
              _full((D, D)), _full((1, D)), _full((D, D))],
    out_specs=[_rows((_RB, D)), _rows((_RB, D))],
    out_shape=(jax.ShapeDtypeStruct((N, D), jnp.float32),
               jax.ShapeDtypeStruct((N, D), jnp.float32)),
)

_tc_post = pl.pallas_call(
    _tc_post_body,
    grid=(N // _RB,),
    in_specs=[_rows((NC, _RB, D)), _rows((_RB, 1)), _rows((_RB, D))],
    out_specs=_rows((_RB, D)),
    out_shape=jax.ShapeDtypeStruct((N, D), jnp.float32),
)


@jax.jit
def kernel(x, edge_index, W_l1, b_l1, W_r1, W_l2, b_l2, W_r2):
    pad = EROWS * D - E
    pidx = jnp.arange(pad, dtype=jnp.int32)
    srcr = jnp.concatenate(
        [edge_index[0].astype(jnp.int32), pidx % N]
    ).reshape(EROWS, D)
    dstr = jnp.concatenate(
        [edge_index[1].astype(jnp.int32), N + pidx % (ACC_N - N)]
    ).reshape(EROWS, D)
    sc_layer1, sc_layer2 = _sc_kernels()
    u1, v1 = _tc_pre(x, W_l1, b_l1.reshape(1, D), W_r1)
    s1, cpart = sc_layer1(u1, srcr, dstr)
    cnt_t = (cpart[0, :N] + cpart[1, :N]).reshape(N, 1)
    u2, v2 = _tc_mid(s1, cnt_t, v1, W_l2, b_l2.reshape(1, D), W_r2)
    s2 = sc_layer2(u2, srcr, dstr)
    return _tc_post(s2, cnt_t, v2)


# speedup vs baseline: 3.6413x; 1.0003x over previous
import functools

import jax
import jax.numpy as jnp
from jax import lax
from jax.experimental import pallas as pl
from jax.experimental.pallas import tpu as pltpu
from jax.experimental.pallas import tpu_sc as plsc

N = 10000
E = 320000
D = 128
NC = 2
NS = 16
NW = NC * NS
RPW = 80
EROWS = NW * RPW
ACC_N = 10240
NPT = 624
NPT1 = ACC_N // NS
CH = 8


def _sc_segsum_body(with_counts, *refs):
    if with_counts:
        (u_hbm, srcr, dstr,
         s_out, c_out,
         s_sh, c_sh, src_v, dst_v, rows_v, ones_v, zc_v, sem, isem) = refs
    else:
        (u_hbm, srcr, dstr,
         s_out,
         s_sh, src_v, dst_v, rows_v, sem, isem) = refs

    cid = lax.axis_index("c")
    sid = lax.axis_index("s")
    wid = cid * NS + sid

    zero16 = jnp.zeros((16,), jnp.float32)

    def zrow(i, carry):
        for k in range(D // 16):
            rows_v[0, i, pl.ds(k * 16, 16)] = zero16
        return carry

    lax.fori_loop(0, D, zrow, 0)
    for b in range(NPT // D):
        pltpu.sync_copy(rows_v.at[0],
                        s_sh.at[pl.ds(sid * NPT + b * D, D)])
    rem = NPT - (NPT // D) * D
    pltpu.sync_copy(rows_v.at[0].at[pl.ds(0, rem)],
                    s_sh.at[pl.ds(sid * NPT + (NPT // D) * D, rem)])
    if with_counts:
        for k in range(NPT1 // 16):
            zc_v[pl.ds(k * 16, 16)] = zero16
        pltpu.sync_copy(zc_v, c_sh.at[pl.ds(sid * NPT1, NPT1)])
        for k in range(D // 16):
            ones_v[pl.ds(k * 16, 16)] = jnp.ones((16,), jnp.float32)

    @pl.when(sid == NS - 1)
    def _():
        tail = ACC_N - NS * NPT
        for b in range(tail // D):
            pltpu.sync_copy(rows_v.at[0],
                            s_sh.at[pl.ds(NS * NPT + b * D, D)])

    base = wid * RPW
    pltpu.sync_copy(srcr.at[pl.ds(base, CH)], src_v.at[0])
    pltpu.sync_copy(dstr.at[pl.ds(base, CH)], dst_v.at[0])

    plsc.subcore_barrier()

    pltpu.async_copy(u_hbm.at[src_v.at[0].at[0]], rows_v.at[0], sem)

    def step(j, carry):
        c = j // CH
        r = j - c * CH

        @pl.when((r == 0) & ((c + 1) * CH < RPW))
        def _():
            pltpu.async_copy(srcr.at[pl.ds(base + (c + 1) * CH, CH)],
                             src_v.at[(c + 1) % 2], isem)
            pltpu.async_copy(dstr.at[pl.ds(base + (c + 1) * CH, CH)],
                             dst_v.at[(c + 1) % 2], isem)

        @pl.when((r == CH - 1) & ((c + 1) * CH < RPW))
        def _():
            pltpu.make_async_copy(srcr.at[pl.ds(base, CH)],
                                  src_v.at[(c + 1) % 2], isem).wait()
            pltpu.make_async_copy(dstr.at[pl.ds(base, CH)],
                                  dst_v.at[(c + 1) % 2], isem).wait()

        nxt = j + 1

        @pl.when(nxt < RPW)
        def _():
            c2 = nxt // CH
            r2 = nxt - c2 * CH
            pltpu.async_copy(u_hbm.at[src_v.at[c2 % 2].at[r2]],
                             rows_v.at[nxt % 2], sem)

        pltpu.make_async_copy(u_hbm.at[src_v.at[c % 2].at[r]],
                              rows_v.at[j % 2], sem).wait()
        pltpu.sync_copy(rows_v.at[j % 2], s_sh.at[dst_v.at[c % 2].at[r]],
                        add=True)
        if with_counts:
            pltpu.sync_copy(ones_v, c_sh.at[dst_v.at[c % 2].at[r]], add=True)
        return carry

    lax.fori_loop(0, RPW, step, 0)

    plsc.subcore_barrier()

    pltpu.sync_copy(s_sh.at[pl.ds(sid * NPT, NPT)],
                    s_out.at[cid].at[pl.ds(sid * NPT, NPT)])
    if with_counts:
        pltpu.sync_copy(c_sh.at[pl.ds(sid * NPT1, NPT1)],
                        c_out.at[cid].at[pl.ds(sid * NPT1, NPT1)])

    @pl.when(sid == NS - 1)
    def _():
        tail = N - NS * NPT
        pltpu.sync_copy(s_sh.at[pl.ds(NS * NPT, tail)],
                        s_out.at[cid].at[pl.ds(NS * NPT, tail)])


@functools.cache
def _sc_kernels():
    mesh = plsc.VectorSubcoreMesh(
        core_axis_name="c", subcore_axis_name="s",
        num_cores=NC, num_subcores=NS,
    )
    layer1 = functools.partial(
        pl.kernel,
        functools.partial(_sc_segsum_body, True),
        out_type=(
            jax.ShapeDtypeStruct((NC, N, D), jnp.float32),
            jax.ShapeDtypeStruct((NC, ACC_N), jnp.float32),
        ),
        mesh=mesh,
        scratch_types=[
            pltpu.VMEM_SHARED((ACC_N, D), jnp.float32),
            pltpu.VMEM_SHARED((ACC_N,), jnp.float32),
            pltpu.VMEM((2, CH, D), jnp.int32),
            pltpu.VMEM((2, CH, D), jnp.int32),
            pltpu.VMEM((2, D, D), jnp.float32),
            pltpu.VMEM((D,), jnp.float32),
            pltpu.VMEM((NPT1,), jnp.float32),
            pltpu.SemaphoreType.DMA,
            pltpu.SemaphoreType.DMA,
        ],
    )()
    layer2 = functools.partial(
        pl.kernel,
        functools.partial(_sc_segsum_body, False),
        out_type=jax.ShapeDtypeStruct((NC, N, D), jnp.float32),
        mesh=mesh,
        scratch_types=[
            pltpu.VMEM_SHARED((ACC_N, D), jnp.float32),
            pltpu.VMEM((2, CH, D), jnp.int32),
            pltpu.VMEM((2, CH, D), jnp.int32),
            pltpu.VMEM((2, D, D), jnp.float32),
            pltpu.SemaphoreType.DMA,
            pltpu.SemaphoreType.DMA,
        ],
    )()
    return layer1, layer2


_RB = 1000


def _tc_pre_body(x_ref, wl_ref, b_ref, wr_ref, u_ref, v_ref):
    x = x_ref[...]
    u_ref[...] = jnp.dot(x, wl_ref[...].T, preferred_element_type=jnp.float32)
    v_ref[...] = (jnp.dot(x, wr_ref[...].T, preferred_element_type=jnp.float32)
                  + b_ref[...])


def _tc_mid_body(s_ref, c_ref, v1_ref, wl_ref, b_ref, wr_ref, u2_ref, v2_ref):
    inv = 1.0 / jnp.maximum(c_ref[...], 1.0)
    h = jnp.maximum((s_ref[0] + s_ref[1]) * inv + v1_ref[...], 0.0)
    u2_ref[...] = jnp.dot(h, wl_ref[...].T, preferred_element_type=jnp.float32)
    v2_ref[...] = (jnp.dot(h, wr_ref[...].T, preferred_element_type=jnp.float32)
                   + b_ref[...])


def _tc_post_body(s_ref, c_ref, v2_ref, out_ref):
    inv = 1.0 / jnp.maximum(c_ref[...], 1.0)
    out_ref[...] = (s_ref[0] + s_ref[1]) * inv + v2_ref[...]


def _full(shape):
    return pl.BlockSpec(shape, lambda i: (0,) * len(shape))


def _rows(shape):
    if len(shape) == 3:
        return pl.BlockSpec(shape, lambda i: (0, i, 0))
    return pl.BlockSpec(shape, lambda i: (i, 0))


_tc_pre = pl.pallas_call(
    _tc_pre_body,
    grid=(N // _RB,),
    in_specs=[_rows((_RB, D)), _full((D, D)), _full((1, D)), _full((D, D))],
    out_specs=[_rows((_RB, D)), _rows((_RB, D))],
    out_shape=(jax.ShapeDtypeStruct((N, D), jnp.float32),
               jax.ShapeDtypeStruct((N, D), jnp.float32)),
)

_tc_mid = pl.pallas_call(
    _tc_mid_body,
    grid=(N // _RB,),
    in_specs=[_rows((NC, _RB, D)), _rows((_RB, 1)), _rows((_RB, D)),
              _full((D, D)), _full((1, D)), _full((D, D))],
    out_specs=[_rows((_RB, D)), _rows((_RB, D))],
    out_shape=(jax.ShapeDtypeStruct((N, D), jnp.float32),
               jax.ShapeDtypeStruct((N, D), jnp.float32)),
)

_tc_post = pl.pallas_call(
    _tc_post_body,
    grid=(N // _RB,),
    in_specs=[_rows((NC, _RB, D)), _rows((_RB, 1)), _rows((_RB, D))],
    out_specs=_rows((_RB, D)),
    out_shape=jax.ShapeDtypeStruct((N, D), jnp.float32),
)


@jax.jit
def kernel(x, edge_index, W_l1, b_l1, W_r1, W_l2, b_l2, W_r2):
    pad = EROWS * D - E
    pidx = jnp.arange(pad, dtype=jnp.int32)
    srcr = jnp.concatenate(
        [edge_index[0].astype(jnp.int32), pidx]
    ).reshape(EROWS, D)
    dstr = jnp.concatenate(
        [edge_index[1].astype(jnp.int32), N + (pidx & 127)]
    ).reshape(EROWS, D)
    sc_layer1, sc_layer2 = _sc_kernels()
    u1, v1 = _tc_pre(x, W_l1, b_l1.reshape(1, D), W_r1)
    s1, cpart = sc_layer1(u1, srcr, dstr)
    cnt_t = (cpart[0, :N] + cpart[1, :N]).reshape(N, 1)
    u2, v2 = _tc_mid(s1, cnt_t, v1, W_l2, b_l2.reshape(1, D), W_r2)
    s2 = sc_layer2(u2, srcr, dstr)
    return _tc_post(s2, cnt_t, v2)


# speedup vs baseline: 3.6520x; 1.0029x over previous
import functools

import jax
import jax.numpy as jnp
from jax import lax
from jax.experimental import pallas as pl
from jax.experimental.pallas import tpu as pltpu
from jax.experimental.pallas import tpu_sc as plsc

N = 10000
E = 320000
D = 128
NC = 2
NS = 16
NW = NC * NS
RPW = 80
EROWS = NW * RPW
ACC_N = 10240
NPT = 624
NPT1 = ACC_N // NS
CH = 8


def _sc_segsum_body(with_counts, *refs):
    if with_counts:
        (u_hbm, srcr, dstr,
         s_out, c_out,
         s_sh, c_sh, src_v, dst_v, rows_v, ones_v, zc_v, sem, isem,
         ssem) = refs
    else:
        (u_hbm, srcr, dstr,
         s_out,
         s_sh, src_v, dst_v, rows_v, sem, isem, ssem) = refs

    cid = lax.axis_index("c")
    sid = lax.axis_index("s")
    wid = cid * NS + sid

    zero16 = jnp.zeros((16,), jnp.float32)

    def zrow(i, carry):
        for k in range(D // 16):
            rows_v[0, i, pl.ds(k * 16, 16)] = zero16
        return carry

    lax.fori_loop(0, D, zrow, 0)
    for b in range(NPT // D):
        pltpu.sync_copy(rows_v.at[0],
                        s_sh.at[pl.ds(sid * NPT + b * D, D)])
    rem = NPT - (NPT // D) * D
    pltpu.sync_copy(rows_v.at[0].at[pl.ds(0, rem)],
                    s_sh.at[pl.ds(sid * NPT + (NPT // D) * D, rem)])
    if with_counts:
        for k in range(NPT1 // 16):
            zc_v[pl.ds(k * 16, 16)] = zero16
        pltpu.sync_copy(zc_v, c_sh.at[pl.ds(sid * NPT1, NPT1)])
        for k in range(D // 16):
            ones_v[pl.ds(k * 16, 16)] = jnp.ones((16,), jnp.float32)

    @pl.when(sid == NS - 1)
    def _():
        tail = ACC_N - NS * NPT
        for b in range(tail // D):
            pltpu.sync_copy(rows_v.at[0],
                            s_sh.at[pl.ds(NS * NPT + b * D, D)])

    base = wid * RPW
    pltpu.sync_copy(srcr.at[pl.ds(base, CH)], src_v.at[0])
    pltpu.sync_copy(dstr.at[pl.ds(base, CH)], dst_v.at[0])

    plsc.subcore_barrier()

    pltpu.async_copy(u_hbm.at[src_v.at[0].at[0]], rows_v.at[0], sem)

    def step(j, carry):
        c = j // CH
        r = j - c * CH

        @pl.when((r == 0) & ((c + 1) * CH < RPW))
        def _():
            pltpu.async_copy(srcr.at[pl.ds(base + (c + 1) * CH, CH)],
                             src_v.at[(c + 1) % 2], isem)
            pltpu.async_copy(dstr.at[pl.ds(base + (c + 1) * CH, CH)],
                             dst_v.at[(c + 1) % 2], isem)

        @pl.when((r == CH - 1) & ((c + 1) * CH < RPW))
        def _():
            pltpu.make_async_copy(srcr.at[pl.ds(base, CH)],
                                  src_v.at[(c + 1) % 2], isem).wait()
            pltpu.make_async_copy(dstr.at[pl.ds(base, CH)],
                                  dst_v.at[(c + 1) % 2], isem).wait()

        @pl.when(j >= 1)
        def _():
            pltpu.make_async_copy(rows_v.at[(j + 1) % 2],
                                  s_sh.at[dst_v.at[c % 2].at[r]], ssem).wait()

        nxt = j + 1

        @pl.when(nxt < RPW)
        def _():
            c2 = nxt // CH
            r2 = nxt - c2 * CH
            pltpu.async_copy(u_hbm.at[src_v.at[c2 % 2].at[r2]],
                             rows_v.at[nxt % 2], sem)

        pltpu.make_async_copy(u_hbm.at[src_v.at[c % 2].at[r]],
                              rows_v.at[j % 2], sem).wait()
        pltpu.async_copy(rows_v.at[j % 2], s_sh.at[dst_v.at[c % 2].at[r]],
                         ssem, add=True)
        if with_counts:
            pltpu.sync_copy(ones_v, c_sh.at[dst_v.at[c % 2].at[r]], add=True)
        return carry

    lax.fori_loop(0, RPW, step, 0)
    pltpu.make_async_copy(rows_v.at[0], s_sh.at[dst_v.at[0].at[0]],
                          ssem).wait()

    plsc.subcore_barrier()

    pltpu.sync_copy(s_sh.at[pl.ds(sid * NPT, NPT)],
                    s_out.at[cid].at[pl.ds(sid * NPT, NPT)])
    if with_counts:
        pltpu.sync_copy(c_sh.at[pl.ds(sid * NPT1, NPT1)],
                        c_out.at[cid].at[pl.ds(sid * NPT1, NPT1)])

    @pl.when(sid == NS - 1)
    def _():
        tail = N - NS * NPT
        pltpu.sync_copy(s_sh.at[pl.ds(NS * NPT, tail)],
                        s_out.at[cid].at[pl.ds(NS * NPT, tail)])


@functools.cache
def _sc_kernels():
    mesh = plsc.VectorSubcoreMesh(
        core_axis_name="c", subcore_axis_name="s",
        num_cores=NC, num_subcores=NS,
    )
    layer1 = functools.partial(
        pl.kernel,
        functools.partial(_sc_segsum_body, True),
        out_type=(
            jax.ShapeDtypeStruct((NC, N, D), jnp.float32),
            jax.ShapeDtypeStruct((NC, ACC_N), jnp.float32),
        ),
        mesh=mesh,
        scratch_types=[
            pltpu.VMEM_SHARED((ACC_N, D), jnp.float32),
            pltpu.VMEM_SHARED((ACC_N,), jnp.float32),
            pltpu.VMEM((2, CH, D), jnp.int32),
            pltpu.VMEM((2, CH, D), jnp.int32),
            pltpu.VMEM((2, D, D), jnp.float32),
            pltpu.VMEM((D,), jnp.float32),
            pltpu.VMEM((NPT1,), jnp.float32),
            pltpu.SemaphoreType.DMA,
            pltpu.SemaphoreType.DMA,
            pltpu.SemaphoreType.DMA,
        ],
    )()
    layer2 = functools.partial(
        pl.kernel,
        functools.partial(_sc_segsum_body, False),
        out_type=jax.ShapeDtypeStruct((NC, N, D), jnp.float32),
        mesh=mesh,
        scratch_types=[
            pltpu.VMEM_SHARED((ACC_N, D), jnp.float32),
            pltpu.VMEM((2, CH, D), jnp.int32),
            pltpu.VMEM((2, CH, D), jnp.int32),
            pltpu.VMEM((2, D, D), jnp.float32),
            pltpu.SemaphoreType.DMA,
            pltpu.SemaphoreType.DMA,
            pltpu.SemaphoreType.DMA,
        ],
    )()
    return layer1, layer2


_RB = 1000


def _tc_pre_body(x_ref, wl_ref, b_ref, wr_ref, u_ref, v_ref):
    x = x_ref[...]
    u_ref[...] = jnp.dot(x, wl_ref[...].T, preferred_element_type=jnp.float32)
    v_ref[...] = (jnp.dot(x, wr_ref[...].T, preferred_element_type=jnp.float32)
                  + b_ref[...])


def _tc_mid_body(s_ref, c_ref, v1_ref, wl_ref, b_ref, wr_ref, u2_ref, v2_ref):
    inv = 1.0 / jnp.maximum(c_ref[...], 1.0)
    h = jnp.maximum((s_ref[0] + s_ref[1]) * inv + v1_ref[...], 0.0)
    u2_ref[...] = jnp.dot(h, wl_ref[...].T, preferred_element_type=jnp.float32)
    v2_ref[...] = (jnp.dot(h, wr_ref[...].T, preferred_element_type=jnp.float32)
                   + b_ref[...])


def _tc_post_body(s_ref, c_ref, v2_ref, out_ref):
    inv = 1.0 / jnp.maximum(c_ref[...], 1.0)
    out_ref[...] = (s_ref[0] + s_ref[1]) * inv + v2_ref[...]


def _full(shape):
    return pl.BlockSpec(shape, lambda i: (0,) * len(shape))


def _rows(shape):
    if len(shape) == 3:
        return pl.BlockSpec(shape, lambda i: (0, i, 0))
    return pl.BlockSpec(shape, lambda i: (i, 0))


_tc_pre = pl.pallas_call(
    _tc_pre_body,
    grid=(N // _RB,),
    in_specs=[_rows((_RB, D)), _full((D, D)), _full((1, D)), _full((D, D))],
    out_specs=[_rows((_RB, D)), _rows((_RB, D))],
    out_shape=(jax.ShapeDtypeStruct((N, D), jnp.float32),
               jax.ShapeDtypeStruct((N, D), jnp.float32)),
)

_tc_mid = pl.pallas_call(
    _tc_mid_body,
    grid=(N // _RB,),
    in_specs=[_rows((NC, _RB, D)), _rows((_RB, 1)), _rows((_RB, D)),
              _full((D, D)), _full((1, D)), _full((D, D))],
    out_specs=[_rows((_RB, D)), _rows((_RB, D))],
    out_shape=(jax.ShapeDtypeStruct((N, D), jnp.float32),
               jax.ShapeDtypeStruct((N, D), jnp.float32)),
)

_tc_post = pl.pallas_call(
    _tc_post_body,
    grid=(N // _RB,),
    in_specs=[_rows((NC, _RB, D)), _rows((_RB, 1)), _rows((_RB, D))],
    out_specs=_rows((_RB, D)),
    out_shape=jax.ShapeDtypeStruct((N, D), jnp.float32),
)


@jax.jit
def kernel(x, edge_index, W_l1, b_l1, W_r1, W_l2, b_l2, W_r2):
    pad = EROWS * D - E
    pidx = jnp.arange(pad, dtype=jnp.int32)
    srcr = jnp.concatenate(
        [edge_index[0].astype(jnp.int32), pidx]
    ).reshape(EROWS, D)
    dstr = jnp.concatenate(
        [edge_index[1].astype(jnp.int32), N + (pidx & 127)]
    ).reshape(EROWS, D)
    sc_layer1, sc_layer2 = _sc_kernels()
    u1, v1 = _tc_pre(x, W_l1, b_l1.reshape(1, D), W_r1)
    s1, cpart = sc_layer1(u1, srcr, dstr)
    cnt_t = (cpart[0, :N] + cpart[1, :N]).reshape(N, 1)
    u2, v2 = _tc_mid(s1, cnt_t, v1, W_l2, b_l2.reshape(1, D), W_r2)
    s2 = sc_layer2(u2, srcr, dstr)
    return _tc_post(s2, cnt_t, v2)


# speedup vs baseline: 3.6656x; 1.0037x over previous
import functools

import jax
import jax.numpy as jnp
from jax import lax
from jax.experimental import pallas as pl
from jax.experimental.pallas import tpu as pltpu
from jax.experimental.pallas import tpu_sc as plsc

N = 10000
E = 320000
D = 128
NC = 2
NS = 16
NW = NC * NS
RPW = 80
EROWS = NW * RPW
ACC_N = 10240
NPT = 624
NPT1 = ACC_N // NS
CH = 8


def _sc_segsum_body(with_counts, *refs):
    if with_counts:
        (u_hbm, srcr, dstr,
         s_out, c_out,
         s_sh, c_sh, src_v, dst_v, rows_v, ones_v, zc_v, sem, isem,
         ssem) = refs
    else:
        (u_hbm, srcr, dstr,
         s_out,
         s_sh, src_v, dst_v, rows_v, sem, isem, ssem) = refs

    cid = lax.axis_index("c")
    sid = lax.axis_index("s")
    wid = cid * NS + sid

    zero16 = jnp.zeros((16,), jnp.float32)

    def zrow(i, carry):
        for k in range(D // 16):
            rows_v[0, i, pl.ds(k * 16, 16)] = zero16
        return carry

    lax.fori_loop(0, D, zrow, 0)
    for b in range(NPT // D):
        pltpu.sync_copy(rows_v.at[0],
                        s_sh.at[pl.ds(sid * NPT + b * D, D)])
    rem = NPT - (NPT // D) * D
    pltpu.sync_copy(rows_v.at[0].at[pl.ds(0, rem)],
                    s_sh.at[pl.ds(sid * NPT + (NPT // D) * D, rem)])
    if with_counts:
        for k in range(NPT1 // 16):
            zc_v[pl.ds(k * 16, 16)] = zero16
        pltpu.sync_copy(zc_v, c_sh.at[pl.ds(sid * NPT1, NPT1)])
        for k in range(D // 16):
            ones_v[pl.ds(k * 16, 16)] = jnp.ones((16,), jnp.float32)

    @pl.when(sid == NS - 1)
    def _():
        tail = ACC_N - NS * NPT
        for b in range(tail // D):
            pltpu.sync_copy(rows_v.at[0],
                            s_sh.at[pl.ds(NS * NPT + b * D, D)])

    base = wid * RPW
    pltpu.sync_copy(srcr.at[pl.ds(base, CH)], src_v.at[0])
    pltpu.sync_copy(dstr.at[pl.ds(base, CH)], dst_v.at[0])

    plsc.subcore_barrier()

    pltpu.async_copy(u_hbm.at[src_v.at[0].at[0]], rows_v.at[0], sem)

    def step(j, carry):
        c = j // CH
        r = j - c * CH

        @pl.when((r == 0) & ((c + 1) * CH < RPW))
        def _():
            pltpu.async_copy(srcr.at[pl.ds(base + (c + 1) * CH, CH)],
                             src_v.at[(c + 1) % 2], isem)
            pltpu.async_copy(dstr.at[pl.ds(base + (c + 1) * CH, CH)],
                             dst_v.at[(c + 1) % 2], isem)

        @pl.when((r == CH - 1) & ((c + 1) * CH < RPW))
        def _():
            pltpu.make_async_copy(srcr.at[pl.ds(base, CH)],
                                  src_v.at[(c + 1) % 2], isem).wait()
            pltpu.make_async_copy(dstr.at[pl.ds(base, CH)],
                                  dst_v.at[(c + 1) % 2], isem).wait()

        @pl.when(j >= 1)
        def _():
            pltpu.make_async_copy(rows_v.at[(j + 1) % 2],
                                  s_sh.at[dst_v.at[c % 2].at[r]], ssem).wait()

        nxt = j + 1

        @pl.when(nxt < RPW)
        def _():
            c2 = nxt // CH
            r2 = nxt - c2 * CH
            pltpu.async_copy(u_hbm.at[src_v.at[c2 % 2].at[r2]],
                             rows_v.at[nxt % 2], sem)

        pltpu.make_async_copy(u_hbm.at[src_v.at[c % 2].at[r]],
                              rows_v.at[j % 2], sem).wait()
        pltpu.async_copy(rows_v.at[j % 2], s_sh.at[dst_v.at[c % 2].at[r]],
                         ssem, add=True)
        if with_counts:
            pltpu.sync_copy(ones_v, c_sh.at[dst_v.at[c % 2].at[r]], add=True)
        return carry

    lax.fori_loop(0, RPW, step, 0)
    pltpu.make_async_copy(rows_v.at[0], s_sh.at[dst_v.at[0].at[0]],
                          ssem).wait()

    plsc.subcore_barrier()

    pltpu.sync_copy(s_sh.at[pl.ds(sid * NPT, NPT)],
                    s_out.at[cid].at[pl.ds(sid * NPT, NPT)])
    if with_counts:
        pltpu.sync_copy(c_sh.at[pl.ds(sid * NPT1, NPT1)],
                        c_out.at[cid].at[pl.ds(sid * NPT1, NPT1)])

    @pl.when(sid == NS - 1)
    def _():
        tail = N - NS * NPT
        pltpu.sync_copy(s_sh.at[pl.ds(NS * NPT, tail)],
                        s_out.at[cid].at[pl.ds(NS * NPT, tail)])


@functools.cache
def _sc_kernels():
    mesh = plsc.VectorSubcoreMesh(
        core_axis_name="c", subcore_axis_name="s",
        num_cores=NC, num_subcores=NS,
    )
    layer1 = functools.partial(
        pl.kernel,
        functools.partial(_sc_segsum_body, True),
        out_type=(
            jax.ShapeDtypeStruct((NC, N, D), jnp.float32),
            jax.ShapeDtypeStruct((NC, ACC_N), jnp.float32),
        ),
        mesh=mesh,
        scratch_types=[
            pltpu.VMEM_SHARED((ACC_N, D), jnp.float32),
            pltpu.VMEM_SHARED((ACC_N,), jnp.float32),
            pltpu.VMEM((2, CH, D), jnp.int32),
            pltpu.VMEM((2, CH, D), jnp.int32),
            pltpu.VMEM((2, D, D), jnp.float32),
            pltpu.VMEM((D,), jnp.float32),
            pltpu.VMEM((NPT1,), jnp.float32),
            pltpu.SemaphoreType.DMA,
            pltpu.SemaphoreType.DMA,
            pltpu.SemaphoreType.DMA,
        ],
    )()
    layer2 = functools.partial(
        pl.kernel,
        functools.partial(_sc_segsum_body, False),
        out_type=jax.ShapeDtypeStruct((NC, N, D), jnp.float32),
        mesh=mesh,
        scratch_types=[
            pltpu.VMEM_SHARED((ACC_N, D), jnp.float32),
            pltpu.VMEM((2, CH, D), jnp.int32),
            pltpu.VMEM((2, CH, D), jnp.int32),
            pltpu.VMEM((2, D, D), jnp.float32),
            pltpu.SemaphoreType.DMA,
            pltpu.SemaphoreType.DMA,
            pltpu.SemaphoreType.DMA,
        ],
    )()
    return layer1, layer2


_RB = 1000


def _tc_pre_u_body(x_ref, wl_ref, u_ref):
    u_ref[...] = jnp.dot(x_ref[...], wl_ref[...].T,
                         preferred_element_type=jnp.float32)


def _tc_pre_v_body(x_ref, wr_ref, b_ref, v_ref):
    v_ref[...] = (jnp.dot(x_ref[...], wr_ref[...].T,
                          preferred_element_type=jnp.float32) + b_ref[...])


def _mid_h(s_ref, c_ref, v1_ref):
    inv = 1.0 / jnp.maximum(c_ref[...], 1.0)
    return jnp.maximum((s_ref[0] + s_ref[1]) * inv + v1_ref[...], 0.0)


def _tc_mid_u_body(s_ref, c_ref, v1_ref, wl_ref, u2_ref):
    h = _mid_h(s_ref, c_ref, v1_ref)
    u2_ref[...] = jnp.dot(h, wl_ref[...].T, preferred_element_type=jnp.float32)


def _tc_mid_v_body(s_ref, c_ref, v1_ref, wr_ref, b_ref, v2_ref):
    h = _mid_h(s_ref, c_ref, v1_ref)
    v2_ref[...] = (jnp.dot(h, wr_ref[...].T,
                           preferred_element_type=jnp.float32) + b_ref[...])


def _tc_post_body(s_ref, c_ref, v2_ref, out_ref):
    inv = 1.0 / jnp.maximum(c_ref[...], 1.0)
    out_ref[...] = (s_ref[0] + s_ref[1]) * inv + v2_ref[...]


def _full(shape):
    return pl.BlockSpec(shape, lambda i: (0,) * len(shape))


def _rows(shape):
    if len(shape) == 3:
        return pl.BlockSpec(shape, lambda i: (0, i, 0))
    return pl.BlockSpec(shape, lambda i: (i, 0))


_nd = jax.ShapeDtypeStruct((N, D), jnp.float32)

_tc_pre_u = pl.pallas_call(
    _tc_pre_u_body, grid=(N // _RB,),
    in_specs=[_rows((_RB, D)), _full((D, D))],
    out_specs=_rows((_RB, D)), out_shape=_nd,
)

_tc_pre_v = pl.pallas_call(
    _tc_pre_v_body, grid=(N // _RB,),
    in_specs=[_rows((_RB, D)), _full((D, D)), _full((1, D))],
    out_specs=_rows((_RB, D)), out_shape=_nd,
)

_tc_mid_u = pl.pallas_call(
    _tc_mid_u_body, grid=(N // _RB,),
    in_specs=[_rows((NC, _RB, D)), _rows((_RB, 1)), _rows((_RB, D)),
              _full((D, D))],
    out_specs=_rows((_RB, D)), out_shape=_nd,
)

_tc_mid_v = pl.pallas_call(
    _tc_mid_v_body, grid=(N // _RB,),
    in_specs=[_rows((NC, _RB, D)), _rows((_RB, 1)), _rows((_RB, D)),
              _full((D, D)), _full((1, D))],
    out_specs=_rows((_RB, D)), out_shape=_nd,
)

_tc_post = pl.pallas_call(
    _tc_post_body,
    grid=(N // _RB,),
    in_specs=[_rows((NC, _RB, D)), _rows((_RB, 1)), _rows((_RB, D))],
    out_specs=_rows((_RB, D)),
    out_shape=jax.ShapeDtypeStruct((N, D), jnp.float32),
)


@jax.jit
def kernel(x, edge_index, W_l1, b_l1, W_r1, W_l2, b_l2, W_r2):
    pad = EROWS * D - E
    pidx = jnp.arange(pad, dtype=jnp.int32)
    srcr = jnp.concatenate(
        [edge_index[0].astype(jnp.int32), pidx]
    ).reshape(EROWS, D)
    dstr = jnp.concatenate(
        [edge_index[1].astype(jnp.int32), N + (pidx & 127)]
    ).reshape(EROWS, D)
    sc_layer1, sc_layer2 = _sc_kernels()
    u1 = _tc_pre_u(x, W_l1)
    s1, cpart = sc_layer1(u1, srcr, dstr)
    v1 = _tc_pre_v(x, W_r1, b_l1.reshape(1, D))
    cnt_t = (cpart[0, :N] + cpart[1, :N]).reshape(N, 1)
    u2 = _tc_mid_u(s1, cnt_t, v1, W_l2)
    s2 = sc_layer2(u2, srcr, dstr)
    v2 = _tc_mid_v(s1, cnt_t, v1, W_r2, b_l2.reshape(1, D))
    return _tc_post(s2, cnt_t, v2)
